# baseline (speedup 1.0000x reference)
import jax
import jax.numpy as jnp
from jax.experimental import pallas as pl
from jax.experimental.pallas import tpu as pltpu

_K = 8192
_D = 32
_COMMIT = 0.25
_M = 128
_H = _K // 2


def _sumsq_tree(x2, axis):
    if axis == 1:
        u = ((x2[:, 0:8] + x2[:, 8:16]) + x2[:, 16:24]) + x2[:, 24:32]
        v = u[:, 0:4] + u[:, 4:8]
        w = v[:, 0:2] + v[:, 2:4]
        return w[:, 0:1] + w[:, 1:2]
    else:
        u = ((x2[0:8, :] + x2[8:16, :]) + x2[16:24, :]) + x2[24:32, :]
        v = u[0:4, :] + u[4:8, :]
        w = v[0:2, :] + v[2:4, :]
        return w[0:1, :] + w[1:2, :]


def _vq_block(f_ref, wt_ref, w_ref, idx_ref, zqst_ref, part_ref):
    wt = wt_ref[...]
    wsq = _sumsq_tree(wt * wt, axis=0)

    f = f_ref[...]
    fsq = _sumsq_tree(f * f, axis=1)
    mm = jnp.dot(f * 2.0, wt, preferred_element_type=jnp.float32)
    dist = (fsq - mm) + wsq

    iota = jax.lax.broadcasted_iota(jnp.int32, (_M, _K), 1)
    d1, d2 = dist[:, :_H], dist[:, _H:]
    m1 = jnp.min(d1, axis=1, keepdims=True)
    m2 = jnp.min(d2, axis=1, keepdims=True)
    i1 = jnp.min(jnp.where(d1 == m1, iota[:, :_H], _K), axis=1)
    i2 = jnp.min(jnp.where(d2 == m2, iota[:, _H:], _K), axis=1)
    m1c, m2c = m1[:, 0], m2[:, 0]
    thr = m1c.astype(jnp.bfloat16).astype(jnp.float32)
    take2 = (m2c < thr) | ((m2c == thr) & (i2 < i1))
    idx = jnp.where(take2, i2, i1)
    idx_ref[...] = idx.reshape(1, 1, _M)

    onehot = (iota == idx[:, None]).astype(jnp.float32)
    zq = jnp.dot(onehot, w_ref[...], preferred_element_type=jnp.float32)
    diff = zq - f
    zqst_ref[...] = f + diff
    part_ref[...] = jnp.broadcast_to(jnp.sum(diff * diff).reshape(1, 1, 1),
                                     (1, 1, 128))


def _loss_reduce(part_ref, out_ref):
    out_ref[...] = (jnp.sum(part_ref[...]) * (1.0 / 128.0)).reshape(1, 1)


def kernel(z, W):
    B, T, D = z.shape
    flat = z.reshape(-1, D)
    n = flat.shape[0]
    nb = n // _M
    wt = W.T

    idx3, zqst, parts = pl.pallas_call(
        _vq_block,
        grid=(nb,),
        in_specs=[
            pl.BlockSpec((_M, D), lambda i: (i, 0)),
            pl.BlockSpec((D, _K), lambda i: (0, 0)),
            pl.BlockSpec((_K, D), lambda i: (0, 0)),
        ],
        out_specs=[
            pl.BlockSpec((1, 1, _M), lambda i: (i, 0, 0)),
            pl.BlockSpec((_M, D), lambda i: (i, 0)),
            pl.BlockSpec((1, 1, 128), lambda i: (i, 0, 0)),
        ],
        out_shape=[
            jax.ShapeDtypeStruct((nb, 1, _M), jnp.int32),
            jax.ShapeDtypeStruct((n, D), jnp.float32),
            jax.ShapeDtypeStruct((nb, 1, 128), jnp.float32),
        ],
        compiler_params=pltpu.CompilerParams(
            dimension_semantics=("parallel",)),
    )(flat, wt, W)

    losssum = pl.pallas_call(
        _loss_reduce,
        in_specs=[pl.BlockSpec((nb, 1, 128), lambda: (0, 0, 0))],
        out_specs=pl.BlockSpec((1, 1), lambda: (0, 0)),
        out_shape=jax.ShapeDtypeStruct((1, 1), jnp.float32),
    )(parts)

    mean = losssum[0, 0] * (1.0 / (n * D))
    loss = mean + _COMMIT * mean
    return (zqst.reshape(z.shape), loss, idx3.reshape(B, T))


# speedup vs baseline: 1.3171x; 1.3171x over previous
import functools

import jax
import jax.numpy as jnp
from jax import lax
from jax.experimental import pallas as pl
from jax.experimental.pallas import tpu as pltpu
from jax.experimental.pallas import tpu_sc as plsc

_K = 8192
_D = 32
_COMMIT = 0.25
_M = 128
_H = _K // 2
_N = 32768


def _sumsq_tree(x2, axis):
    if axis == 1:
        u = ((x2[:, 0:8] + x2[:, 8:16]) + x2[:, 16:24]) + x2[:, 24:32]
        v = u[:, 0:4] + u[:, 4:8]
        w = v[:, 0:2] + v[:, 2:4]
        return w[:, 0:1] + w[:, 1:2]
    else:
        u = ((x2[0:8, :] + x2[8:16, :]) + x2[16:24, :]) + x2[24:32, :]
        v = u[0:4, :] + u[4:8, :]
        w = v[0:2, :] + v[2:4, :]
        return w[0:1, :] + w[1:2, :]


def _argmin_block(f_ref, wt_ref, idx_ref, wsq_ref):
    b = pl.program_id(0)

    @pl.when(b == 0)
    def _init():
        wt0 = wt_ref[...]
        wsq_ref[...] = _sumsq_tree(wt0 * wt0, axis=0)

    f = f_ref[...]
    fsq = _sumsq_tree(f * f, axis=1)
    mm = jnp.dot(f * 2.0, wt_ref[...], preferred_element_type=jnp.float32)
    dist = (fsq - mm) + wsq_ref[...]

    iota = lax.broadcasted_iota(jnp.int32, (_M, _K), 1)
    d1, d2 = dist[:, :_H], dist[:, _H:]
    m1 = jnp.min(d1, axis=1, keepdims=True)
    m2 = jnp.min(d2, axis=1, keepdims=True)
    i1 = jnp.min(jnp.where(d1 == m1, iota[:, :_H], _K), axis=1)
    i2 = jnp.min(jnp.where(d2 == m2, iota[:, _H:], _K), axis=1)
    m1c, m2c = m1[:, 0], m2[:, 0]
    thr = m1c.astype(jnp.bfloat16).astype(jnp.float32)
    take2 = (m2c < thr) | ((m2c == thr) & (i2 < i1))
    idx = jnp.where(take2, i2, i1)
    idx_ref[...] = idx.reshape(1, 1, _M)


_SC_INFO = plsc.get_sparse_core_info()
_NW = _SC_INFO.num_cores * _SC_INFO.num_subcores
_BPW = _N // _NW


_DP = 128


_CH = _BPW // 2


def _sc_gather(table_hbm, idx_hbm, out_hbm, idx_v, rows_v, sem):
    wid = lax.axis_index("s") * _SC_INFO.num_cores + lax.axis_index("c")
    base = wid * _BPW
    for c in range(_BPW // _CH):
        lo = base + c * _CH
        pltpu.sync_copy(idx_hbm.at[pl.ds(lo, _CH)], idx_v)
        pltpu.async_copy(table_hbm.at[idx_v], rows_v, sem).wait()
        pltpu.sync_copy(rows_v, out_hbm.at[pl.ds(lo, _CH)])


def _st_loss_block(f_ref, zq_ref, zqst_ref, loss_ref, acc_ref):
    b = pl.program_id(0)

    @pl.when(b == 0)
    def _init():
        acc_ref[...] = jnp.zeros((1, 1), jnp.float32)

    f = f_ref[...]
    diff = zq_ref[...][:, :_D] - f
    zqst_ref[...] = f + diff
    acc_ref[...] += jnp.sum(diff * diff).reshape(1, 1)

    @pl.when(b == pl.num_programs(0) - 1)
    def _fin():
        loss_ref[...] = acc_ref[...]


def kernel(z, W):
    B, T, D = z.shape
    flat = z.reshape(-1, D)
    n = flat.shape[0]
    nb = n // _M
    wt = W.T

    idx3 = pl.pallas_call(
        _argmin_block,
        grid=(nb,),
        in_specs=[
            pl.BlockSpec((_M, D), lambda i: (i, 0)),
            pl.BlockSpec((D, _K), lambda i: (0, 0)),
        ],
        out_specs=pl.BlockSpec((1, 1, _M), lambda i: (i, 0, 0)),
        out_shape=jax.ShapeDtypeStruct((nb, 1, _M), jnp.int32),
        scratch_shapes=[pltpu.VMEM((1, _K), jnp.float32)],
    )(flat, wt)
    idx_flat = idx3.reshape(-1)

    wpad = jnp.pad(W, ((0, 0), (0, _DP - D)))
    gather = functools.partial(
        pl.kernel,
        mesh=plsc.VectorSubcoreMesh(core_axis_name="c", subcore_axis_name="s"),
        out_type=jax.ShapeDtypeStruct((n, _DP), jnp.float32),
        scratch_types=[
            pltpu.VMEM((_CH,), jnp.int32),
            pltpu.VMEM((_CH, _DP), jnp.float32),
            pltpu.SemaphoreType.DMA,
        ],
    )(_sc_gather)
    zq = gather(wpad, idx_flat)

    mrows = n // 8
    zqst, losssum = pl.pallas_call(
        _st_loss_block,
        grid=(8,),
        in_specs=[
            pl.BlockSpec((mrows, D), lambda i: (i, 0)),
            pl.BlockSpec((mrows, _DP), lambda i: (i, 0)),
        ],
        out_specs=[
            pl.BlockSpec((mrows, D), lambda i: (i, 0)),
            pl.BlockSpec((1, 1), lambda i: (0, 0)),
        ],
        out_shape=[
            jax.ShapeDtypeStruct((n, D), jnp.float32),
            jax.ShapeDtypeStruct((1, 1), jnp.float32),
        ],
        scratch_shapes=[pltpu.VMEM((1, 1), jnp.float32)],
    )(flat, zq)

    mean = losssum[0, 0] * (1.0 / (n * D))
    loss = mean + _COMMIT * mean
    return (zqst.reshape(z.shape), loss, idx3.reshape(B, T))


# speedup vs baseline: 1.4498x; 1.1008x over previous
import functools

import jax
import jax.numpy as jnp
from jax import lax
from jax.experimental import pallas as pl
from jax.experimental.pallas import tpu as pltpu
from jax.experimental.pallas import tpu_sc as plsc

_K = 8192
_D = 32
_COMMIT = 0.25
_M = 256
_H = _K // 2
_N = 32768


def _sumsq_tree(x2, axis):
    if axis == 1:
        u = ((x2[:, 0:8] + x2[:, 8:16]) + x2[:, 16:24]) + x2[:, 24:32]
        v = u[:, 0:4] + u[:, 4:8]
        w = v[:, 0:2] + v[:, 2:4]
        return w[:, 0:1] + w[:, 1:2]
    else:
        u = ((x2[0:8, :] + x2[8:16, :]) + x2[16:24, :]) + x2[24:32, :]
        v = u[0:4, :] + u[4:8, :]
        w = v[0:2, :] + v[2:4, :]
        return w[0:1, :] + w[1:2, :]


def _argmin_block(f_ref, wt_ref, idx_ref, wsq_ref):
    b = pl.program_id(0)

    @pl.when(b == 0)
    def _init():
        wt0 = wt_ref[...]
        wsq_ref[...] = _sumsq_tree(wt0 * wt0, axis=0)

    f = f_ref[...]
    fsq = _sumsq_tree(f * f, axis=1)
    mm = jnp.dot(f * 2.0, wt_ref[...], preferred_element_type=jnp.float32)
    dist = (fsq - mm) + wsq_ref[...]

    iota = lax.broadcasted_iota(jnp.int32, (_M, _K), 1)
    d1, d2 = dist[:, :_H], dist[:, _H:]
    m1 = jnp.min(d1, axis=1, keepdims=True)
    m2 = jnp.min(d2, axis=1, keepdims=True)
    i1 = jnp.min(jnp.where(d1 == m1, iota[:, :_H], _K), axis=1)
    i2 = jnp.min(jnp.where(d2 == m2, iota[:, _H:], _K), axis=1)
    m1c, m2c = m1[:, 0], m2[:, 0]
    thr = m1c.astype(jnp.bfloat16).astype(jnp.float32)
    take2 = (m2c < thr) | ((m2c == thr) & (i2 < i1))
    idx = jnp.where(take2, i2, i1)
    idx_ref[...] = idx.reshape(1, 1, _M)


_SC_INFO = plsc.get_sparse_core_info()
_NW = _SC_INFO.num_cores * _SC_INFO.num_subcores
_BPW = _N // _NW


_DP = 128


_CH = _BPW // 2


def _sc_gather(table_hbm, idx_hbm, out_hbm, idx_v, rows_v, sem):
    wid = lax.axis_index("s") * _SC_INFO.num_cores + lax.axis_index("c")
    base = wid * _BPW
    for c in range(_BPW // _CH):
        lo = base + c * _CH
        pltpu.sync_copy(idx_hbm.at[pl.ds(lo, _CH)], idx_v)
        pltpu.async_copy(table_hbm.at[idx_v], rows_v, sem).wait()
        pltpu.sync_copy(rows_v, out_hbm.at[pl.ds(lo, _CH)])


def _st_loss_block(f_ref, zq_ref, zqst_ref, loss_ref, acc_ref):
    b = pl.program_id(0)

    @pl.when(b == 0)
    def _init():
        acc_ref[...] = jnp.zeros((1, 1), jnp.float32)

    f = f_ref[...]
    diff = zq_ref[...][:, :_D] - f
    zqst_ref[...] = f + diff
    acc_ref[...] += jnp.sum(diff * diff).reshape(1, 1)

    @pl.when(b == pl.num_programs(0) - 1)
    def _fin():
        loss_ref[...] = acc_ref[...]


def kernel(z, W):
    B, T, D = z.shape
    flat = z.reshape(-1, D)
    n = flat.shape[0]
    nb = n // _M
    wt = W.T

    idx3 = pl.pallas_call(
        _argmin_block,
        grid=(nb,),
        in_specs=[
            pl.BlockSpec((_M, D), lambda i: (i, 0)),
            pl.BlockSpec((D, _K), lambda i: (0, 0)),
        ],
        out_specs=pl.BlockSpec((1, 1, _M), lambda i: (i, 0, 0)),
        out_shape=jax.ShapeDtypeStruct((nb, 1, _M), jnp.int32),
        scratch_shapes=[pltpu.VMEM((1, _K), jnp.float32)],
    )(flat, wt)
    idx_flat = idx3.reshape(-1)

    wpad = jnp.pad(W, ((0, 0), (0, _DP - D)))
    gather = functools.partial(
        pl.kernel,
        mesh=plsc.VectorSubcoreMesh(core_axis_name="c", subcore_axis_name="s"),
        out_type=jax.ShapeDtypeStruct((n, _DP), jnp.float32),
        scratch_types=[
            pltpu.VMEM((_CH,), jnp.int32),
            pltpu.VMEM((_CH, _DP), jnp.float32),
            pltpu.SemaphoreType.DMA,
        ],
    )(_sc_gather)
    zq = gather(wpad, idx_flat)

    mrows = n // 8
    zqst, losssum = pl.pallas_call(
        _st_loss_block,
        grid=(8,),
        in_specs=[
            pl.BlockSpec((mrows, D), lambda i: (i, 0)),
            pl.BlockSpec((mrows, _DP), lambda i: (i, 0)),
        ],
        out_specs=[
            pl.BlockSpec((mrows, D), lambda i: (i, 0)),
            pl.BlockSpec((1, 1), lambda i: (0, 0)),
        ],
        out_shape=[
            jax.ShapeDtypeStruct((n, D), jnp.float32),
            jax.ShapeDtypeStruct((1, 1), jnp.float32),
        ],
        scratch_shapes=[pltpu.VMEM((1, 1), jnp.float32)],
    )(flat, zq)

    mean = losssum[0, 0] * (1.0 / (n * D))
    loss = mean + _COMMIT * mean
    return (zqst.reshape(z.shape), loss, idx3.reshape(B, T))


# speedup vs baseline: 1.5104x; 1.0418x over previous
import functools

import jax
import jax.numpy as jnp
from jax import lax
from jax.experimental import pallas as pl
from jax.experimental.pallas import tpu as pltpu
from jax.experimental.pallas import tpu_sc as plsc

_K = 8192
_D = 32
_COMMIT = 0.25
_M = 512
_H = _K // 2
_N = 32768


def _sumsq_tree(x2, axis):
    if axis == 1:
        u = ((x2[:, 0:8] + x2[:, 8:16]) + x2[:, 16:24]) + x2[:, 24:32]
        v = u[:, 0:4] + u[:, 4:8]
        w = v[:, 0:2] + v[:, 2:4]
        return w[:, 0:1] + w[:, 1:2]
    else:
        u = ((x2[0:8, :] + x2[8:16, :]) + x2[16:24, :]) + x2[24:32, :]
        v = u[0:4, :] + u[4:8, :]
        w = v[0:2, :] + v[2:4, :]
        return w[0:1, :] + w[1:2, :]


def _argmin_block(f_ref, wt_ref, idx_ref, wsq_ref):
    b = pl.program_id(0)

    @pl.when(b == 0)
    def _init():
        wt0 = wt_ref[...]
        wsq_ref[...] = _sumsq_tree(wt0 * wt0, axis=0)

    f = f_ref[...]
    fsq = _sumsq_tree(f * f, axis=1)
    mm = jnp.dot(f * 2.0, wt_ref[...], preferred_element_type=jnp.float32)
    dist = (fsq - mm) + wsq_ref[...]

    iota = lax.broadcasted_iota(jnp.int32, (_M, _K), 1)
    d1, d2 = dist[:, :_H], dist[:, _H:]
    m1 = jnp.min(d1, axis=1, keepdims=True)
    m2 = jnp.min(d2, axis=1, keepdims=True)
    i1 = jnp.min(jnp.where(d1 == m1, iota[:, :_H], _K), axis=1)
    i2 = jnp.min(jnp.where(d2 == m2, iota[:, _H:], _K), axis=1)
    m1c, m2c = m1[:, 0], m2[:, 0]
    thr = m1c.astype(jnp.bfloat16).astype(jnp.float32)
    take2 = (m2c < thr) | ((m2c == thr) & (i2 < i1))
    idx = jnp.where(take2, i2, i1)
    idx_ref[...] = idx.reshape(1, 1, _M)


_SC_INFO = plsc.get_sparse_core_info()
_NW = _SC_INFO.num_cores * _SC_INFO.num_subcores
_BPW = _N // _NW


_DP = 128


_CH = _BPW // 2


def _sc_gather(table_hbm, idx_hbm, out_hbm, idx_v, rows_v, sem):
    wid = lax.axis_index("s") * _SC_INFO.num_cores + lax.axis_index("c")
    base = wid * _BPW
    for c in range(_BPW // _CH):
        lo = base + c * _CH
        pltpu.sync_copy(idx_hbm.at[pl.ds(lo, _CH)], idx_v)
        pltpu.async_copy(table_hbm.at[idx_v], rows_v, sem).wait()
        pltpu.sync_copy(rows_v, out_hbm.at[pl.ds(lo, _CH)])


def _st_loss_block(f_ref, zq_ref, zqst_ref, loss_ref, acc_ref):
    b = pl.program_id(0)

    @pl.when(b == 0)
    def _init():
        acc_ref[...] = jnp.zeros((1, 1), jnp.float32)

    f = f_ref[...]
    diff = zq_ref[...][:, :_D] - f
    zqst_ref[...] = f + diff
    acc_ref[...] += jnp.sum(diff * diff).reshape(1, 1)

    @pl.when(b == pl.num_programs(0) - 1)
    def _fin():
        loss_ref[...] = acc_ref[...]


def kernel(z, W):
    B, T, D = z.shape
    flat = z.reshape(-1, D)
    n = flat.shape[0]
    nb = n // _M
    wt = W.T

    idx3 = pl.pallas_call(
        _argmin_block,
        grid=(nb,),
        in_specs=[
            pl.BlockSpec((_M, D), lambda i: (i, 0)),
            pl.BlockSpec((D, _K), lambda i: (0, 0)),
        ],
        out_specs=pl.BlockSpec((1, 1, _M), lambda i: (i, 0, 0)),
        out_shape=jax.ShapeDtypeStruct((nb, 1, _M), jnp.int32),
        scratch_shapes=[pltpu.VMEM((1, _K), jnp.float32)],
    )(flat, wt)
    idx_flat = idx3.reshape(-1)

    wpad = jnp.pad(W, ((0, 0), (0, _DP - D)))
    gather = functools.partial(
        pl.kernel,
        mesh=plsc.VectorSubcoreMesh(core_axis_name="c", subcore_axis_name="s"),
        out_type=jax.ShapeDtypeStruct((n, _DP), jnp.float32),
        scratch_types=[
            pltpu.VMEM((_CH,), jnp.int32),
            pltpu.VMEM((_CH, _DP), jnp.float32),
            pltpu.SemaphoreType.DMA,
        ],
    )(_sc_gather)
    zq = gather(wpad, idx_flat)

    mrows = n // 8
    zqst, losssum = pl.pallas_call(
        _st_loss_block,
        grid=(8,),
        in_specs=[
            pl.BlockSpec((mrows, D), lambda i: (i, 0)),
            pl.BlockSpec((mrows, _DP), lambda i: (i, 0)),
        ],
        out_specs=[
            pl.BlockSpec((mrows, D), lambda i: (i, 0)),
            pl.BlockSpec((1, 1), lambda i: (0, 0)),
        ],
        out_shape=[
            jax.ShapeDtypeStruct((n, D), jnp.float32),
            jax.ShapeDtypeStruct((1, 1), jnp.float32),
        ],
        scratch_shapes=[pltpu.VMEM((1, 1), jnp.float32)],
    )(flat, zq)

    mean = losssum[0, 0] * (1.0 / (n * D))
    loss = mean + _COMMIT * mean
    return (zqst.reshape(z.shape), loss, idx3.reshape(B, T))


# speedup vs baseline: 1.7231x; 1.1408x over previous
import functools

import jax
import jax.numpy as jnp
from jax import lax
from jax.experimental import pallas as pl
from jax.experimental.pallas import tpu as pltpu
from jax.experimental.pallas import tpu_sc as plsc

_K = 8192
_D = 32
_COMMIT = 0.25
_M = 512
_H = _K // 2
_N = 32768
_CW = 256


def _sumsq_tree(x2, axis):
    if axis == 1:
        u = ((x2[:, 0:8] + x2[:, 8:16]) + x2[:, 16:24]) + x2[:, 24:32]
        v = u[:, 0:4] + u[:, 4:8]
        w = v[:, 0:2] + v[:, 2:4]
        return w[:, 0:1] + w[:, 1:2]
    else:
        u = ((x2[0:8, :] + x2[8:16, :]) + x2[16:24, :]) + x2[24:32, :]
        v = u[0:4, :] + u[4:8, :]
        w = v[0:2, :] + v[2:4, :]
        return w[0:1, :] + w[1:2, :]


def _argmin_block(f_ref, wt_ref, idx_ref, wsq_ref):
    b = pl.program_id(0)

    @pl.when(b == 0)
    def _init():
        wt0 = wt_ref[...]
        wsq_ref[...] = _sumsq_tree(wt0 * wt0, axis=0)

    f = f_ref[...]
    fsq = _sumsq_tree(f * f, axis=1)
    f2 = f * 2.0

    def half_argmin(c_lo, c_hi):
        rv = jnp.full((_M, _CW), jnp.inf, jnp.float32)
        rt = jnp.zeros((_M, _CW), jnp.int32)
        for c in range(c_lo, c_hi):
            sl = pl.ds(c * _CW, _CW)
            mmc = jnp.dot(f2, wt_ref[:, sl], preferred_element_type=jnp.float32)
            dc = (fsq - mmc) + wsq_ref[:, sl]
            lt = dc < rv
            rv = jnp.where(lt, dc, rv)
            rt = jnp.where(lt, c, rt)
        m = jnp.min(rv, axis=1, keepdims=True)
        lane = lax.broadcasted_iota(jnp.int32, (_M, _CW), 1)
        idxc = rt * _CW + lane
        i = jnp.min(jnp.where(rv == m, idxc, _K), axis=1)
        return m[:, 0], i

    nch = _K // _CW
    m1c, i1 = half_argmin(0, nch // 2)
    m2c, i2 = half_argmin(nch // 2, nch)
    thr = m1c.astype(jnp.bfloat16).astype(jnp.float32)
    take2 = (m2c < thr) | ((m2c == thr) & (i2 < i1))
    idx = jnp.where(take2, i2, i1)
    idx_ref[...] = idx.reshape(1, 1, _M)


_SC_INFO = plsc.get_sparse_core_info()
_NW = _SC_INFO.num_cores * _SC_INFO.num_subcores
_BPW = _N // _NW


_DP = 128


_CH = _BPW // 2


def _sc_gather(table_hbm, idx_hbm, out_hbm, idx_v, rows_v, sem):
    wid = lax.axis_index("s") * _SC_INFO.num_cores + lax.axis_index("c")
    base = wid * _BPW
    for c in range(_BPW // _CH):
        lo = base + c * _CH
        pltpu.sync_copy(idx_hbm.at[pl.ds(lo, _CH)], idx_v)
        pltpu.async_copy(table_hbm.at[idx_v], rows_v, sem).wait()
        pltpu.sync_copy(rows_v, out_hbm.at[pl.ds(lo, _CH)])


def _st_loss_block(f_ref, zq_ref, zqst_ref, loss_ref, acc_ref):
    b = pl.program_id(0)

    @pl.when(b == 0)
    def _init():
        acc_ref[...] = jnp.zeros((1, 1), jnp.float32)

    f = f_ref[...]
    diff = zq_ref[...][:, :_D] - f
    zqst_ref[...] = f + diff
    acc_ref[...] += jnp.sum(diff * diff).reshape(1, 1)

    @pl.when(b == pl.num_programs(0) - 1)
    def _fin():
        loss_ref[...] = acc_ref[...]


def kernel(z, W):
    B, T, D = z.shape
    flat = z.reshape(-1, D)
    n = flat.shape[0]
    nb = n // _M
    wt = W.T

    idx3 = pl.pallas_call(
        _argmin_block,
        grid=(nb,),
        in_specs=[
            pl.BlockSpec((_M, D), lambda i: (i, 0)),
            pl.BlockSpec((D, _K), lambda i: (0, 0)),
        ],
        out_specs=pl.BlockSpec((1, 1, _M), lambda i: (i, 0, 0)),
        out_shape=jax.ShapeDtypeStruct((nb, 1, _M), jnp.int32),
        scratch_shapes=[pltpu.VMEM((1, _K), jnp.float32)],
    )(flat, wt)
    idx_flat = idx3.reshape(-1)

    wpad = jnp.pad(W, ((0, 0), (0, _DP - D)))
    gather = functools.partial(
        pl.kernel,
        mesh=plsc.VectorSubcoreMesh(core_axis_name="c", subcore_axis_name="s"),
        out_type=jax.ShapeDtypeStruct((n, _DP), jnp.float32),
        scratch_types=[
            pltpu.VMEM((_CH,), jnp.int32),
            pltpu.VMEM((_CH, _DP), jnp.float32),
            pltpu.SemaphoreType.DMA,
        ],
    )(_sc_gather)
    zq = gather(wpad, idx_flat)

    mrows = n // 8
    zqst, losssum = pl.pallas_call(
        _st_loss_block,
        grid=(8,),
        in_specs=[
            pl.BlockSpec((mrows, D), lambda i: (i, 0)),
            pl.BlockSpec((mrows, _DP), lambda i: (i, 0)),
        ],
        out_specs=[
            pl.BlockSpec((mrows, D), lambda i: (i, 0)),
            pl.BlockSpec((1, 1), lambda i: (0, 0)),
        ],
        out_shape=[
            jax.ShapeDtypeStruct((n, D), jnp.float32),
            jax.ShapeDtypeStruct((1, 1), jnp.float32),
        ],
        scratch_shapes=[pltpu.VMEM((1, 1), jnp.float32)],
    )(flat, zq)

    mean = losssum[0, 0] * (1.0 / (n * D))
    loss = mean + _COMMIT * mean
    return (zqst.reshape(z.shape), loss, idx3.reshape(B, T))
